# Initial kernel scaffold; baseline (speedup 1.0000x reference)
import functools
import math

import jax
import jax.numpy as jnp
from jax import lax
from jax.experimental import pallas as pl
from jax.experimental.pallas import tpu as pltpu
from jax.experimental.pallas import tpu_sc as plsc

N = 16384 * 200
D = 32
SCALE = math.sqrt(32.0)

_info = plsc.get_sparse_core_info()
NC = _info.num_cores
NS = _info.num_subcores
NW = NC * NS
L = _info.num_lanes

PER_W = N // NW
CHUNK = 1024
GATHER = 128
K = CHUNK // GATHER
NCHUNK = PER_W // CHUNK

_mesh = plsc.VectorSubcoreMesh(core_axis_name="c", subcore_axis_name="s")


@functools.partial(
    pl.kernel,
    mesh=_mesh,
    out_type=jax.ShapeDtypeStruct((N, D), jnp.float32),
    scratch_types=[
        pltpu.VMEM((K, GATHER), jnp.int32),
        pltpu.VMEM((CHUNK, D), jnp.float32),
        pltpu.SemaphoreType.DMA,
    ],
)
def _embed(x_hbm, table_hbm, out_hbm, idx_v, rows_v, sem):
    wid = lax.axis_index("s") * NC + lax.axis_index("c")
    wrow0 = wid * (PER_W // GATHER)

    def chunk_body(ci, carry):
        row0 = wrow0 + ci * K
        base = row0 * GATHER
        pltpu.sync_copy(x_hbm.at[pl.ds(row0, K)], idx_v)
        copies = []
        for j in range(K):
            copies.append(
                pltpu.async_copy(
                    table_hbm.at[idx_v.at[j]],
                    rows_v.at[pl.ds(j * GATHER, GATHER)],
                    sem,
                )
            )
        for c in copies:
            c.wait()

        def scale_body(i, carry2):
            r = i // 2
            h = (i % 2) * L
            rows_v[r, pl.ds(h, L)] = rows_v[r, pl.ds(h, L)] * SCALE
            return carry2

        lax.fori_loop(0, CHUNK * D // L, scale_body, 0, unroll=8)

        pltpu.sync_copy(rows_v, out_hbm.at[pl.ds(base, CHUNK)])
        return carry

    lax.fori_loop(0, NCHUNK, chunk_body, 0)


def kernel(x, table):
    x_flat = x.reshape(N // GATHER, GATHER)
    out = _embed(x_flat, table)
    return out.reshape(16384, 200, D)


# speedup vs baseline: 4.5669x; 4.5669x over previous
import functools
import math

import jax
import jax.numpy as jnp
from jax import lax
from jax.experimental import pallas as pl
from jax.experimental.pallas import tpu as pltpu
from jax.experimental.pallas import tpu_sc as plsc

N = 16384 * 200
D = 32
SCALE = math.sqrt(32.0)

_info = plsc.get_sparse_core_info()
NC = _info.num_cores
NS = _info.num_subcores
NW = NC * NS
L = _info.num_lanes

PER_W = N // NW
CHUNK = 1024
GATHER = 128
K = CHUNK // GATHER
NCHUNK = PER_W // CHUNK

_mesh = plsc.VectorSubcoreMesh(core_axis_name="c", subcore_axis_name="s")


@functools.partial(
    pl.kernel,
    mesh=_mesh,
    out_type=jax.ShapeDtypeStruct((N, D), jnp.float32),
    scratch_types=[
        pltpu.VMEM((K, GATHER), jnp.int32),
        pltpu.VMEM((CHUNK, D), jnp.float32),
        pltpu.SemaphoreType.DMA,
    ],
    compiler_params=pltpu.CompilerParams(use_tc_tiling_on_sc=False),
)
def _embed(x_hbm, table_hbm, out_hbm, idx_v, rows_v, sem):
    wid = lax.axis_index("s") * NC + lax.axis_index("c")
    wrow0 = wid * (PER_W // GATHER)

    def chunk_body(ci, carry):
        row0 = wrow0 + ci * K
        base = row0 * GATHER
        pltpu.sync_copy(x_hbm.at[pl.ds(row0, K)], idx_v)
        copies = []
        for j in range(K):
            copies.append(
                pltpu.async_copy(
                    table_hbm.at[idx_v.at[j]],
                    rows_v.at[pl.ds(j * GATHER, GATHER)],
                    sem,
                )
            )
        for c in copies:
            c.wait()

        def scale_body(i, carry2):
            r = i // 2
            h = (i % 2) * L
            rows_v[r, pl.ds(h, L)] = rows_v[r, pl.ds(h, L)] * SCALE
            return carry2

        lax.fori_loop(0, CHUNK * D // L, scale_body, 0, unroll=8)

        pltpu.sync_copy(rows_v, out_hbm.at[pl.ds(base, CHUNK)])
        return carry

    lax.fori_loop(0, NCHUNK, chunk_body, 0)


def kernel(x, table):
    x_flat = x.reshape(N // GATHER, GATHER)
    out = _embed(x_flat, table)
    return out.reshape(16384, 200, D)


# speedup vs baseline: 4.5674x; 1.0001x over previous
import functools
import math

import jax
import jax.numpy as jnp
from jax import lax
from jax.experimental import pallas as pl
from jax.experimental.pallas import tpu as pltpu
from jax.experimental.pallas import tpu_sc as plsc

N = 16384 * 200
D = 32
SCALE = math.sqrt(32.0)

_info = plsc.get_sparse_core_info()
NC = _info.num_cores
NS = _info.num_subcores
NW = NC * NS
L = _info.num_lanes

PER_W = N // NW
CHUNK = 1024
GATHER = 128
K = CHUNK // GATHER
NCHUNK = PER_W // CHUNK

_mesh = plsc.VectorSubcoreMesh(core_axis_name="c", subcore_axis_name="s")


@functools.partial(
    pl.kernel,
    mesh=_mesh,
    out_type=jax.ShapeDtypeStruct((N, D), jnp.float32),
    scratch_types=[
        pltpu.VMEM((K, GATHER), jnp.int32),
        pltpu.VMEM((CHUNK, D), jnp.float32),
        pltpu.SemaphoreType.DMA,
    ],
    compiler_params=pltpu.CompilerParams(use_tc_tiling_on_sc=False),
)
def _embed(x_hbm, table_hbm, out_hbm, idx_v, rows_v, sem):
    wid = lax.axis_index("s") * NC + lax.axis_index("c")
    wrow0 = wid * (PER_W // GATHER)

    def chunk_body(ci, carry):
        row0 = wrow0 + ci * K
        base = row0 * GATHER
        pltpu.sync_copy(x_hbm.at[pl.ds(row0, K)], idx_v)
        copies = []
        for j in range(K):
            copies.append(
                pltpu.async_copy(
                    table_hbm.at[idx_v.at[j]],
                    rows_v.at[pl.ds(j * GATHER, GATHER)],
                    sem,
                )
            )
        for c in copies:
            c.wait()

        def scale_body(r, carry2):
            rows_v[r, pl.ds(0, L)] = rows_v[r, pl.ds(0, L)] * SCALE
            rows_v[r, pl.ds(L, L)] = rows_v[r, pl.ds(L, L)] * SCALE
            return carry2

        lax.fori_loop(0, CHUNK, scale_body, 0, unroll=8)

        pltpu.sync_copy(rows_v, out_hbm.at[pl.ds(base, CHUNK)])
        return carry

    lax.fori_loop(0, NCHUNK, chunk_body, 0)


def kernel(x, table):
    x_flat = x.reshape(N // GATHER, GATHER)
    out = _embed(x_flat, table)
    return out.reshape(16384, 200, D)
